# baseline (speedup 1.0000x reference)
import functools

import jax
import jax.numpy as jnp
from jax import lax
from jax.experimental import pallas as pl
from jax.experimental.pallas import tpu as pltpu
from jax.experimental.pallas import tpu_sc as plsc

_B, _T, _N, _C = 16, 4096, 64, 4
_PS = 2
_CB = _T // _PS
_R = _B * _CB
_W = _PS * _N * _C
_NW = 32
_RPW = _R // _NW
_CHUNK = 64
_NCH = _RPW // _CHUNK

_mesh = plsc.VectorSubcoreMesh(core_axis_name="c", subcore_axis_name="s")


@functools.partial(
    pl.kernel,
    out_type=jax.ShapeDtypeStruct((_R, _W), jnp.float32),
    mesh=_mesh,
    scratch_types=[
        pltpu.VMEM((_CHUNK,), jnp.int32),
        pltpu.VMEM((_CHUNK, _W), jnp.float32),
        pltpu.VMEM((_CHUNK, _W), jnp.float32),
        pltpu.SemaphoreType.DMA,
    ],
)
def _patch_perm(x_hbm, perm_hbm, out_hbm, idx_v, a_v, b_v, sem):
    cid = lax.axis_index("c")
    sid = lax.axis_index("s")
    wid = sid * 2 + cid
    base_row = wid * _RPW
    row_off = (base_row // _CB) * _CB
    ch0 = (lax.iota(jnp.int32, 16) % _C) == 0

    def chunk_body(ci, carry):
        r0 = base_row + ci * _CHUNK
        pltpu.sync_copy(perm_hbm.at[pl.ds(r0, _CHUNK)], idx_v)

        def add_off(k, c):
            idx_v[pl.ds(k * 16, 16)] = idx_v[pl.ds(k * 16, 16)] + row_off
            return c

        lax.fori_loop(0, _CHUNK // 16, add_off, 0, unroll=True)

        gat = pltpu.async_copy(x_hbm.at[idx_v], a_v, sem)
        pltpu.sync_copy(x_hbm.at[pl.ds(r0, _CHUNK)], b_v)
        gat.wait()

        def blend_row(r, c):
            for col in range(_W // 16):
                sl = pl.ds(col * 16, 16)
                a = a_v[r, sl]
                b = b_v[r, sl]
                b_v[r, sl] = jnp.where(ch0, a, b)
            return c

        lax.fori_loop(0, _CHUNK, blend_row, 0)
        pltpu.sync_copy(b_v, out_hbm.at[pl.ds(r0, _CHUNK)])
        return carry

    lax.fori_loop(0, _NCH, chunk_body, 0)


def kernel(input, perm):
    x2 = input.reshape(_R, _W)
    p1 = perm.reshape(_R)
    out2 = _patch_perm(x2, p1)
    return out2.reshape(_B, _T, _N, _C)


# speedup vs baseline: 1.0013x; 1.0013x over previous
import functools

import jax
import jax.numpy as jnp
from jax import lax
from jax.experimental import pallas as pl
from jax.experimental.pallas import tpu as pltpu
from jax.experimental.pallas import tpu_sc as plsc

_B, _T, _N, _C = 16, 4096, 64, 4
_PS = 2
_CB = _T // _PS
_R = _B * _CB
_W = _PS * _N * _C
_LR = _W // 128
_R3 = _R * _LR
_NW = 32
_RPW = _R // _NW
_CHUNK = 32
_CL = _CHUNK * _LR
_NCH = _RPW // _CHUNK

_mesh = plsc.VectorSubcoreMesh(core_axis_name="c", subcore_axis_name="s")


@functools.partial(
    pl.kernel,
    out_type=jax.ShapeDtypeStruct((_R3, 128), jnp.float32),
    mesh=_mesh,
    compiler_params=pltpu.CompilerParams(needs_layout_passes=False),
    scratch_types=[
        pltpu.VMEM((_CHUNK,), jnp.int32),
        pltpu.VMEM((_CL,), jnp.int32),
        pltpu.VMEM((_CL, 128), jnp.float32),
        pltpu.VMEM((_CL, 128), jnp.float32),
        pltpu.SemaphoreType.DMA,
    ],
)
def _patch_perm(x_hbm, perm_hbm, out_hbm, pv_v, idx_v, a_v, b_v, sem):
    cid = lax.axis_index("c")
    sid = lax.axis_index("s")
    wid = sid * 2 + cid
    base_row = wid * _RPW
    row_off = (base_row // _CB) * _CB
    lane = lax.iota(jnp.int32, 16)
    ch0 = (lane % _C) == 0

    def chunk_body(ci, carry):
        j0 = base_row + ci * _CHUNK
        r3 = j0 * _LR
        pltpu.sync_copy(perm_hbm.at[pl.ds(j0, _CHUNK)], pv_v)

        for w in range(_CHUNK // 16):
            pv = pv_v[pl.ds(w * 16, 16)]
            val = (pv + row_off) * _LR
            tgt = (w * 16 + lane) * _LR
            for k in range(_LR):
                plsc.store_scatter(idx_v, [tgt + k], val + k)

        gat = pltpu.async_copy(x_hbm.at[idx_v], a_v, sem)
        pltpu.sync_copy(x_hbm.at[pl.ds(r3, _CL)], b_v)
        gat.wait()

        def blend_row(r, c):
            for col in range(128 // 16):
                sl = pl.ds(col * 16, 16)
                a = a_v[r, sl]
                b = b_v[r, sl]
                b_v[r, sl] = jnp.where(ch0, a, b)
            return c

        lax.fori_loop(0, _CL, blend_row, 0)
        pltpu.sync_copy(b_v, out_hbm.at[pl.ds(r3, _CL)])
        return carry

    lax.fori_loop(0, _NCH, chunk_body, 0)


def kernel(input, perm):
    x3 = input.reshape(_R3, 128)
    p1 = perm.reshape(_R)
    out3 = _patch_perm(x3, p1)
    return out3.reshape(_B, _T, _N, _C)


# speedup vs baseline: 75.9784x; 75.8810x over previous
import functools

import jax
import jax.numpy as jnp
from jax import lax
from jax.experimental import pallas as pl
from jax.experimental.pallas import tpu as pltpu
from jax.experimental.pallas import tpu_sc as plsc

_B, _T, _N, _C = 16, 4096, 64, 4
_PS = 2
_CB = _T // _PS
_TT = _T // 128
_PANEL = _TT * _C * 128
_TOT = _B * _T * _N * _C
_NPW = _N // 2
_PPC = 2
_CH = _PPC * _PANEL
_NCHUNK = _NPW // _PPC

_mesh = plsc.VectorSubcoreMesh(core_axis_name="c", subcore_axis_name="s")


@functools.partial(
    pl.kernel,
    out_type=jax.ShapeDtypeStruct((_TOT,), jnp.float32),
    mesh=_mesh,
    compiler_params=pltpu.CompilerParams(needs_layout_passes=False),
    scratch_types=[
        pltpu.VMEM((_CB,), jnp.int32),
        pltpu.VMEM((_T,), jnp.int32),
        pltpu.VMEM((_CH,), jnp.float32),
        pltpu.VMEM((_CH,), jnp.float32),
        pltpu.VMEM((_T,), jnp.float32),
        pltpu.SemaphoreType.DMA,
        pltpu.SemaphoreType.DMA,
        pltpu.SemaphoreType.DMA,
        pltpu.SemaphoreType.DMA,
    ],
)
def _patch_perm(x_hbm, perm_hbm, out_hbm, pv_v, g2f_v, buf_a, buf_b,
                c0_v, isem_a, isem_b, osem_a, osem_b):
    cid = lax.axis_index("c")
    sid = lax.axis_index("s")
    wid = sid * 2 + cid
    b = wid // 2
    n0 = (wid % 2) * _NPW
    lane = lax.iota(jnp.int32, 16)
    base0 = (b * _N + n0) * _PANEL

    pltpu.sync_copy(perm_hbm.at[pl.ds(b * _CB, _CB)], pv_v)

    def build_idx(w, carry):
        p = pv_v[pl.ds(w * 16, 16)]
        f0 = (p // 64) * 512 + ((p * 2) % 128)
        t0 = (w * 16 + lane) * 2
        plsc.store_scatter(g2f_v, [t0], f0)
        plsc.store_scatter(g2f_v, [t0 + 1], f0 + 1)
        return carry

    lax.fori_loop(0, _CB // 16, build_idx, 0)

    bufs = (buf_a, buf_b)
    isems = (isem_a, isem_b)
    osems = (osem_a, osem_b)

    def start_in(i):
        return pltpu.async_copy(
            x_hbm.at[pl.ds(base0 + i * _CH, _CH)], bufs[i % 2], isems[i % 2])

    def start_out(i):
        return pltpu.async_copy(
            bufs[i % 2], out_hbm.at[pl.ds(base0 + i * _CH, _CH)],
            osems[i % 2])

    def process(buf, poff):
        def gather16(u, c):
            for k in range(8):
                o = pl.ds((u * 8 + k) * 16, 16)
                idx = g2f_v[o] + poff
                c0_v[o] = plsc.load_gather(buf, [idx])
            return c

        lax.fori_loop(0, _T // 128, gather16, 0)

        def wb(tt, c):
            for k in range(8):
                src = pl.ds(tt * 128 + k * 16, 16)
                dst = pl.ds(poff + tt * 512 + k * 16, 16)
                buf[dst] = c0_v[src]
            return c

        lax.fori_loop(0, _TT, wb, 0)

    incopies = {0: start_in(0)}
    outcopies = {}
    for i in range(_NCHUNK):
        if i + 1 < _NCHUNK:
            if i - 1 >= 0:
                outcopies[i - 1].wait()
            incopies[i + 1] = start_in(i + 1)
        incopies[i].wait()
        for p in range(_PPC):
            process(bufs[i % 2], p * _PANEL)
        outcopies[i] = start_out(i)
    outcopies[_NCHUNK - 2].wait()
    outcopies[_NCHUNK - 1].wait()


def kernel(input, perm):
    t1 = jnp.transpose(input, (0, 2, 3, 1))
    t2 = t1.reshape(_B, _N, _C, _TT, 128)
    t3 = jnp.transpose(t2, (0, 1, 3, 2, 4))
    x1 = t3.reshape(_TOT)
    p1 = perm.reshape(_B * _CB)
    o1 = _patch_perm(x1, p1)
    o3 = o1.reshape(_B, _N, _TT, _C, 128)
    o2 = jnp.transpose(o3, (0, 1, 3, 2, 4))
    ot = o2.reshape(_B, _N, _C, _T)
    return jnp.transpose(ot, (0, 3, 1, 2))


# speedup vs baseline: 117.6381x; 1.5483x over previous
import functools

import jax
import jax.numpy as jnp
from jax import lax
from jax.experimental import pallas as pl
from jax.experimental.pallas import tpu as pltpu
from jax.experimental.pallas import tpu_sc as plsc

_B, _T, _N, _C = 16, 4096, 64, 4
_PS = 2
_CB = _T // _PS
_TT = _T // 128
_PANEL = _TT * _C * 128
_TOT = _B * _T * _N * _C
_NPW = _N // 2
_PPC = 2
_CH = _PPC * _PANEL
_NCHUNK = _NPW // _PPC

_mesh = plsc.VectorSubcoreMesh(core_axis_name="c", subcore_axis_name="s")


@functools.partial(
    pl.kernel,
    out_type=jax.ShapeDtypeStruct((_TOT,), jnp.float32),
    mesh=_mesh,
    compiler_params=pltpu.CompilerParams(needs_layout_passes=False),
    scratch_types=[
        pltpu.VMEM((_CB,), jnp.int32),
        pltpu.VMEM((_CB + 16,), jnp.int32),
        pltpu.VMEM((_CB + 16,), jnp.int32),
        pltpu.VMEM((_CH,), jnp.float32),
        pltpu.VMEM((_CH,), jnp.float32),
        pltpu.VMEM((_T,), jnp.float32),
        pltpu.SemaphoreType.DMA,
        pltpu.SemaphoreType.DMA,
        pltpu.SemaphoreType.DMA,
        pltpu.SemaphoreType.DMA,
    ],
)
def _patch_perm(x_hbm, perm_hbm, out_hbm, pv_v, srcb_v, dstb_v, buf_a, buf_b,
                st_v, isem_a, isem_b, osem_a, osem_b):
    cid = lax.axis_index("c")
    sid = lax.axis_index("s")
    wid = sid * 2 + cid
    b = wid // 2
    n0 = (wid % 2) * _NPW
    lane = lax.iota(jnp.int32, 16)
    base0 = (b * _N + n0) * _PANEL

    pltpu.sync_copy(perm_hbm.at[pl.ds(b * _CB, _CB)], pv_v)

    def build_idx(w, off):
        p = pv_v[pl.ds(w * 16, 16)]
        j = w * 16 + lane
        m = p != j
        mi = jnp.where(m, 1, 0)
        pos = off + plsc.cumsum(mi) - mi
        fsrc = (p // 64) * 512 + ((p * 2) % 128)
        fdst = (j // 64) * 512 + ((j * 2) % 128)
        plsc.store_scatter(srcb_v, [pos], fsrc, mask=m)
        plsc.store_scatter(dstb_v, [pos], fdst, mask=m)
        return off + jnp.sum(mi)

    nmod = lax.fori_loop(0, _CB // 16, build_idx, 0)
    nit = (nmod + 15) // 16

    bufs = (buf_a, buf_b)
    isems = (isem_a, isem_b)
    osems = (osem_a, osem_b)

    def start_in(i):
        return pltpu.async_copy(
            x_hbm.at[pl.ds(base0 + i * _CH, _CH)], bufs[i % 2], isems[i % 2])

    def start_out(i):
        return pltpu.async_copy(
            bufs[i % 2], out_hbm.at[pl.ds(base0 + i * _CH, _CH)],
            osems[i % 2])

    def process(buf, poff):
        def gather(u, c):
            o = pl.ds(u * 16, 16)
            tm = lane < (nmod - u * 16)
            s = srcb_v[o] + poff
            st_v[o] = plsc.load_gather(buf, [s], mask=tm)
            st_v[pl.ds(_CB + u * 16, 16)] = plsc.load_gather(
                buf, [s + 1], mask=tm)
            return c

        lax.fori_loop(0, nit, gather, 0)

        def scatter(u, c):
            o = pl.ds(u * 16, 16)
            tm = lane < (nmod - u * 16)
            d = dstb_v[o] + poff
            plsc.store_scatter(buf, [d], st_v[o], mask=tm)
            plsc.store_scatter(
                buf, [d + 1], st_v[pl.ds(_CB + u * 16, 16)], mask=tm)
            return c

        lax.fori_loop(0, nit, scatter, 0)

    incopies = {0: start_in(0)}
    outcopies = {}
    for i in range(_NCHUNK):
        if i + 1 < _NCHUNK:
            if i - 1 >= 0:
                outcopies[i - 1].wait()
            incopies[i + 1] = start_in(i + 1)
        incopies[i].wait()
        for p in range(_PPC):
            process(bufs[i % 2], p * _PANEL)
        outcopies[i] = start_out(i)
    outcopies[_NCHUNK - 2].wait()
    outcopies[_NCHUNK - 1].wait()


def kernel(input, perm):
    t1 = jnp.transpose(input, (0, 2, 3, 1))
    t2 = t1.reshape(_B, _N, _C, _TT, 128)
    t3 = jnp.transpose(t2, (0, 1, 3, 2, 4))
    x1 = t3.reshape(_TOT)
    p1 = perm.reshape(_B * _CB)
    o1 = _patch_perm(x1, p1)
    o3 = o1.reshape(_B, _N, _TT, _C, 128)
    o2 = jnp.transpose(o3, (0, 1, 3, 2, 4))
    ot = o2.reshape(_B, _N, _C, _T)
    return jnp.transpose(ot, (0, 3, 1, 2))
